# baseline (speedup 1.0000x reference)
import functools

import jax
import jax.numpy as jnp
from jax.experimental import pallas as pl
from jax.experimental.pallas import tpu as pltpu

_VMEM_LIMIT = 110 * 1024 * 1024


def _bf(v):
    return v.astype(jnp.bfloat16)


def _make_mega_kernel(tm, nb):
    def _mega(x_ref, a_ref, w1_ref, w2_ref, o_ref,
              u_ref, m_ref, z_ref, zt_ref):
        s = pl.program_id(0)

        @pl.when(s < nb)
        def _phase_a():
            u_ref[pl.ds(s * tm, tm), :] = _bf(
                jnp.dot(_bf(x_ref[...]), _bf(w1_ref[...]),
                        preferred_element_type=jnp.float32))

        @pl.when((s >= nb) & (s < 2 * nb))
        def _phase_b():
            j = s - nb
            h = jnp.dot(_bf(a_ref[...]), u_ref[...],
                        preferred_element_type=jnp.float32)
            h = jnp.maximum(h, 0.0)
            m_ref[pl.ds(j * tm, tm), :] = _bf(
                jnp.dot(_bf(h), w2_ref[...],
                        preferred_element_type=jnp.float32))

        @pl.when((s >= 2 * nb) & (s < 3 * nb))
        def _phase_c():
            j = s - 2 * nb
            z_ref[pl.ds(j * tm, tm), :] = _bf(
                jnp.dot(_bf(a_ref[...]), m_ref[...],
                        preferred_element_type=jnp.float32))

        @pl.when(s >= 3 * nb)
        def _phase_d():
            i = s - 3 * nb

            @pl.when(s == 3 * nb)
            def _():
                zt_ref[...] = z_ref[...].T

            o_ref[...] = jnp.dot(z_ref[pl.ds(i * tm, tm), :], zt_ref[...],
                                 preferred_element_type=jnp.float32)

    return _mega


def _row_tile(m):
    for t in (200, 100, 80, 40, 16, 8):
        if m % t == 0:
            return t
    return m


@functools.partial(jax.jit, static_argnames=())
def kernel(x, adj, W1, W2):
    n, d = x.shape
    h_dim = W1.shape[1]
    l_dim = W2.shape[1]
    f32 = jnp.float32
    bf16 = jnp.bfloat16

    tm = _row_tile(n)
    nb = n // tm

    def x_map(s):
        return (jnp.minimum(s, nb - 1), 0)

    def a_map(s):
        return (jnp.where(s < nb, 0,
                jnp.where(s < 2 * nb, s - nb,
                jnp.where(s < 3 * nb, s - 2 * nb, nb - 1))), 0)

    def o_map(s):
        return (jnp.where(s < 3 * nb, 0, s - 3 * nb), 0)

    out = pl.pallas_call(
        _make_mega_kernel(tm, nb),
        grid=(4 * nb,),
        in_specs=[
            pl.BlockSpec((tm, d), x_map),
            pl.BlockSpec((tm, n), a_map),
            pl.BlockSpec((d, h_dim), lambda s: (0, 0)),
            pl.BlockSpec((h_dim, l_dim), lambda s: (0, 0)),
        ],
        out_specs=pl.BlockSpec((tm, n), o_map),
        out_shape=jax.ShapeDtypeStruct((n, n), f32),
        scratch_shapes=[
            pltpu.VMEM((n, h_dim), bf16),
            pltpu.VMEM((n, l_dim), bf16),
            pltpu.VMEM((n, l_dim), bf16),
            pltpu.VMEM((l_dim, n), bf16),
        ],
        compiler_params=pltpu.CompilerParams(
            dimension_semantics=("arbitrary",),
            vmem_limit_bytes=_VMEM_LIMIT,
        ),
    )(x, adj, W1, W2)

    return out


# speedup vs baseline: 1.0769x; 1.0769x over previous
import functools

import jax
import jax.numpy as jnp
from jax.experimental import pallas as pl
from jax.experimental.pallas import tpu as pltpu

_VMEM_LIMIT = 100 * 1024 * 1024


def _bf(v):
    return v.astype(jnp.bfloat16)


def _make_enc_kernel(tm, nb):
    def _enc(x_ref, a_ref, w1_ref, w2_ref, z_ref, u_ref, m_ref):
        s = pl.program_id(0)

        @pl.when(s < nb)
        def _phase_a():
            u_ref[pl.ds(s * tm, tm), :] = _bf(
                jnp.dot(_bf(x_ref[...]), _bf(w1_ref[...]),
                        preferred_element_type=jnp.float32))

        @pl.when((s >= nb) & (s < 2 * nb))
        def _phase_b():
            j = s - nb
            h = jnp.dot(_bf(a_ref[...]), u_ref[...],
                        preferred_element_type=jnp.float32)
            h = jnp.maximum(h, 0.0)
            m_ref[pl.ds(j * tm, tm), :] = _bf(
                jnp.dot(_bf(h), w2_ref[...],
                        preferred_element_type=jnp.float32))

        @pl.when(s >= 2 * nb)
        def _phase_c():
            z_ref[...] = _bf(
                jnp.dot(_bf(a_ref[...]), m_ref[...],
                        preferred_element_type=jnp.float32))

    return _enc


def _dec_kernel(z_ref, z_all_ref, o_ref, zt_ref):
    @pl.when(pl.program_id(0) == 0)
    def _():
        zt_ref[...] = z_all_ref[...].T

    o_ref[...] = jnp.dot(z_ref[...], zt_ref[...],
                         preferred_element_type=jnp.float32)


def _row_tile(m):
    for t in (400, 200, 100, 80, 40, 16, 8):
        if m % t == 0:
            return t
    return m


@functools.partial(jax.jit, static_argnames=())
def kernel(x, adj, W1, W2):
    n, d = x.shape
    h_dim = W1.shape[1]
    l_dim = W2.shape[1]
    f32 = jnp.float32
    bf16 = jnp.bfloat16

    tm = _row_tile(n)
    nb = n // tm

    def x_map(s):
        return (jnp.minimum(s, nb - 1), 0)

    def a_map(s):
        return (jnp.where(s < nb, 0,
                jnp.where(s < 2 * nb, s - nb, s - 2 * nb)), 0)

    def z_map(s):
        return (jnp.where(s < 2 * nb, 0, s - 2 * nb), 0)

    z = pl.pallas_call(
        _make_enc_kernel(tm, nb),
        grid=(3 * nb,),
        in_specs=[
            pl.BlockSpec((tm, d), x_map),
            pl.BlockSpec((tm, n), a_map),
            pl.BlockSpec((d, h_dim), lambda s: (0, 0)),
            pl.BlockSpec((h_dim, l_dim), lambda s: (0, 0)),
        ],
        out_specs=pl.BlockSpec((tm, l_dim), z_map),
        out_shape=jax.ShapeDtypeStruct((n, l_dim), bf16),
        scratch_shapes=[
            pltpu.VMEM((n, h_dim), bf16),
            pltpu.VMEM((n, l_dim), bf16),
        ],
        compiler_params=pltpu.CompilerParams(
            dimension_semantics=("arbitrary",),
            vmem_limit_bytes=_VMEM_LIMIT,
        ),
    )(x, adj, W1, W2)

    out = pl.pallas_call(
        _dec_kernel,
        grid=(nb,),
        in_specs=[
            pl.BlockSpec((tm, l_dim), lambda i: (i, 0)),
            pl.BlockSpec((n, l_dim), lambda i: (0, 0)),
        ],
        out_specs=pl.BlockSpec((tm, n), lambda i: (i, 0)),
        out_shape=jax.ShapeDtypeStruct((n, n), f32),
        scratch_shapes=[pltpu.VMEM((l_dim, n), bf16)],
        compiler_params=pltpu.CompilerParams(
            dimension_semantics=("arbitrary",),
            vmem_limit_bytes=_VMEM_LIMIT,
        ),
    )(z, z)

    return out


# speedup vs baseline: 1.1082x; 1.0291x over previous
import functools

import jax
import jax.numpy as jnp
from jax.experimental import pallas as pl
from jax.experimental.pallas import tpu as pltpu

_VMEM_LIMIT = 100 * 1024 * 1024


def _bf(v):
    return v.astype(jnp.bfloat16)


def _make_enc_kernel(tm, tma, nb, nba):
    def _enc(x_ref, a_ref, w1_ref, w2_ref, z_ref, u_ref, m_ref):
        s = pl.program_id(0)

        @pl.when(s < nba)
        def _phase_a():
            u_ref[pl.ds(s * tma, tma), :] = _bf(
                jnp.dot(_bf(x_ref[...]), _bf(w1_ref[...]),
                        preferred_element_type=jnp.float32))

        @pl.when((s >= nba) & (s < nba + nb))
        def _phase_b():
            j = s - nba
            h = jnp.dot(_bf(a_ref[...]), u_ref[...],
                        preferred_element_type=jnp.float32)
            h = jnp.maximum(h, 0.0)
            m_ref[pl.ds(j * tm, tm), :] = _bf(
                jnp.dot(_bf(h), w2_ref[...],
                        preferred_element_type=jnp.float32))

        @pl.when(s >= nba + nb)
        def _phase_c():
            z_ref[...] = _bf(
                jnp.dot(_bf(a_ref[...]), m_ref[...],
                        preferred_element_type=jnp.float32))

    return _enc


def _make_dec_kernel(tm):
    def _dec(z_all_ref, o_ref, zt_ref):
        i = pl.program_id(0)

        @pl.when(i == 0)
        def _():
            zt_ref[...] = z_all_ref[...].T

        o_ref[...] = jnp.dot(z_all_ref[pl.ds(i * tm, tm), :], zt_ref[...],
                             preferred_element_type=jnp.float32)

    return _dec


def _row_tile(m):
    for t in (400, 200, 100, 80, 40, 16, 8):
        if m % t == 0:
            return t
    return m


@functools.partial(jax.jit, static_argnames=())
def kernel(x, adj, W1, W2):
    n, d = x.shape
    h_dim = W1.shape[1]
    l_dim = W2.shape[1]
    f32 = jnp.float32
    bf16 = jnp.bfloat16

    tm = _row_tile(n)
    nb = n // tm
    tma = next((t for t in (2000, 1000, 400, 200, 8) if n % t == 0), n)
    nba = n // tma

    def x_map(s):
        return (jnp.minimum(s, nba - 1), 0)

    def a_map(s):
        return (jnp.where(s < nba, 0,
                jnp.where(s < nba + nb, s - nba, s - nba - nb)), 0)

    def z_map(s):
        return (jnp.where(s < nba + nb, 0, s - nba - nb), 0)

    z = pl.pallas_call(
        _make_enc_kernel(tm, tma, nb, nba),
        grid=(nba + 2 * nb,),
        in_specs=[
            pl.BlockSpec((tma, d), x_map),
            pl.BlockSpec((tm, n), a_map),
            pl.BlockSpec((d, h_dim), lambda s: (0, 0)),
            pl.BlockSpec((h_dim, l_dim), lambda s: (0, 0)),
        ],
        out_specs=pl.BlockSpec((tm, l_dim), z_map),
        out_shape=jax.ShapeDtypeStruct((n, l_dim), bf16),
        scratch_shapes=[
            pltpu.VMEM((n, h_dim), bf16),
            pltpu.VMEM((n, l_dim), bf16),
        ],
        compiler_params=pltpu.CompilerParams(
            dimension_semantics=("arbitrary",),
            vmem_limit_bytes=_VMEM_LIMIT,
        ),
    )(x, adj, W1, W2)

    out = pl.pallas_call(
        _make_dec_kernel(tm),
        grid=(nb,),
        in_specs=[
            pl.BlockSpec((n, l_dim), lambda i: (0, 0)),
        ],
        out_specs=pl.BlockSpec((tm, n), lambda i: (i, 0)),
        out_shape=jax.ShapeDtypeStruct((n, n), f32),
        scratch_shapes=[pltpu.VMEM((l_dim, n), bf16)],
        compiler_params=pltpu.CompilerParams(
            dimension_semantics=("arbitrary",),
            vmem_limit_bytes=_VMEM_LIMIT,
        ),
    )(z)

    return out
